# Initial kernel scaffold; baseline (speedup 1.0000x reference)
import jax
import jax.numpy as jnp
from jax.experimental import pallas as pl


def kernel(input_feature, adj_indices, adj_values, weight):
    raise NotImplementedError("write your pallas kernel here")


# speedup vs baseline: 5.3168x; 5.3168x over previous
import jax
import jax.numpy as jnp
from jax import lax
from jax.experimental import pallas as pl
from jax.experimental.pallas import tpu as pltpu
from jax.experimental.pallas import tpu_sc as plsc

N_NODES = 10000
N_EDGES = 320000
IN_F = 128
OUT_F = 128
NSC = 2
NTILES = 16
NWORK = NSC * NTILES
LANES = 16

CHUNK = 128
N_CHUNKS = N_EDGES // CHUNK
BASE_CHUNKS = N_CHUNKS // NWORK
EXTRA = N_CHUNKS - BASE_CHUNKS * NWORK

ROW_BLK = 1000
ZROWS = 632
TAIL_ROWS = N_NODES - (NTILES - 1) * ZROWS


def _support_body(x_ref, w_ref, o_ref):
    o_ref[...] = jnp.dot(x_ref[...], w_ref[...],
                         preferred_element_type=jnp.float32)


def _compute_support(x, w):
    return pl.pallas_call(
        _support_body,
        grid=(N_NODES // ROW_BLK,),
        in_specs=[
            pl.BlockSpec((ROW_BLK, IN_F), lambda r: (r, 0)),
            pl.BlockSpec((IN_F, OUT_F), lambda r: (0, 0)),
        ],
        out_specs=pl.BlockSpec((ROW_BLK, OUT_F), lambda r: (r, 0)),
        out_shape=jax.ShapeDtypeStruct((N_NODES, OUT_F), jnp.float32),
    )(x, w)


def _add_body(a_ref, b_ref, o_ref):
    o_ref[...] = a_ref[...] + b_ref[...]


def _combine(a, b):
    return pl.pallas_call(
        _add_body,
        grid=(N_NODES // ROW_BLK,),
        in_specs=[pl.BlockSpec((ROW_BLK, OUT_F), lambda r: (r, 0))] * 2,
        out_specs=pl.BlockSpec((ROW_BLK, OUT_F), lambda r: (r, 0)),
        out_shape=jax.ShapeDtypeStruct((N_NODES, OUT_F), jnp.float32),
    )(a, b)


def _sc_body(support, src, dst, val, zeros, out0, out1,
             src_v, dst_v, val_v, rows_v, acc, sem):
    c = lax.axis_index("c")
    s = lax.axis_index("s")
    w = s * NSC + c

    @pl.when(s < NTILES - 1)
    def _():
        pltpu.sync_copy(zeros, acc.at[pl.ds(s * ZROWS, ZROWS)])

    @pl.when(s == NTILES - 1)
    def _():
        pltpu.sync_copy(zeros.at[pl.ds(0, TAIL_ROWS)],
                        acc.at[pl.ds((NTILES - 1) * ZROWS, TAIL_ROWS)])

    plsc.subcore_barrier()

    nk = jnp.where(w < EXTRA, BASE_CHUNKS + 1, BASE_CHUNKS)

    def chunk_body(k, carry):
        base = (w + k * NWORK) * CHUNK
        pltpu.sync_copy(src.at[pl.ds(base, CHUNK)], src_v)
        pltpu.sync_copy(dst.at[pl.ds(base, CHUNK)], dst_v)
        pltpu.sync_copy(val.at[pl.ds(base, CHUNK)], val_v)
        pltpu.async_copy(support.at[src_v], rows_v, sem).wait()
        for g in range(CHUNK // LANES):
            vv = val_v[pl.ds(g * LANES, LANES)]
            for l in range(LANES):
                bv = lax.gather(
                    vv, jnp.full((LANES, 1), l, jnp.int32),
                    dimension_numbers=lax.GatherDimensionNumbers(
                        offset_dims=(), collapsed_slice_dims=(0,),
                        start_index_map=(0,)),
                    slice_sizes=(1,),
                    mode=lax.GatherScatterMode.PROMISE_IN_BOUNDS)
                j = g * LANES + l
                for q in range(OUT_F // LANES):
                    qs = pl.ds(q * LANES, LANES)
                    rows_v[j, qs] = rows_v[j, qs] * bv
        pltpu.sync_copy(rows_v, acc.at[dst_v], add=True)
        return carry

    lax.fori_loop(0, nk, chunk_body, 0)

    plsc.subcore_barrier()

    for ci, out in ((0, out0), (1, out1)):
        @pl.when(c == ci)
        def _():
            @pl.when(s < NTILES - 1)
            def _():
                r0 = s * ZROWS
                pltpu.sync_copy(acc.at[pl.ds(r0, ZROWS)],
                                out.at[pl.ds(r0, ZROWS)])

            @pl.when(s == NTILES - 1)
            def _():
                r0 = (NTILES - 1) * ZROWS
                pltpu.sync_copy(acc.at[pl.ds(r0, TAIL_ROWS)],
                                out.at[pl.ds(r0, TAIL_ROWS)])


_sc_spmm = pl.kernel(
    _sc_body,
    out_type=(jax.ShapeDtypeStruct((N_NODES, OUT_F), jnp.float32),
              jax.ShapeDtypeStruct((N_NODES, OUT_F), jnp.float32)),
    mesh=plsc.VectorSubcoreMesh(core_axis_name="c", subcore_axis_name="s",
                                num_cores=NSC, num_subcores=NTILES),
    scratch_types=[
        pltpu.VMEM((CHUNK,), jnp.int32),
        pltpu.VMEM((CHUNK,), jnp.int32),
        pltpu.VMEM((CHUNK,), jnp.float32),
        pltpu.VMEM((CHUNK, OUT_F), jnp.float32),
        pltpu.VMEM_SHARED((N_NODES, OUT_F), jnp.float32),
        pltpu.SemaphoreType.DMA,
    ],
)


@jax.jit
def kernel(input_feature, adj_indices, adj_values, weight):
    support = _compute_support(input_feature, weight)
    idx = adj_indices.astype(jnp.int32)
    dst = idx[0]
    src = idx[1]
    zeros = jnp.zeros((ZROWS, OUT_F), jnp.float32)
    p0, p1 = _sc_spmm(support, src, dst, adj_values, zeros)
    return _combine(p0, p1)
